# baseline (speedup 1.0000x reference)
import jax
import jax.numpy as jnp
from jax import lax
from jax.experimental import pallas as pl
from jax.experimental.pallas import tpu as pltpu
from jax.experimental.pallas import tpu_sc as plsc

_NC = 2
_NS = 16
_NW = _NC * _NS
_L = 16
_CHUNK = 256


def _sc_body(gum_hbm, gim_hbm, out_hbm, a_v, b_v, o_v):
    rows = o_v.shape[0]
    chunk = a_v.shape[0]
    n_cols = a_v.shape[1]
    wid = lax.axis_index("s") * _NC + lax.axis_index("c")
    base = wid * rows

    lanes = lax.iota(jnp.int32, _L)

    for h in range(rows // chunk):
        pltpu.sync_copy(gum_hbm.at[pl.ds(base + h * chunk, chunk), :], a_v)
        pltpu.sync_copy(gim_hbm.at[pl.ds(base + h * chunk, chunk), :], b_v)

        def group(g, carry):
            row_idx = g * _L + lanes
            acc = jnp.zeros((_L,), jnp.float32)
            for c in range(n_cols):
                col_idx = jnp.full((_L,), c, jnp.int32)
                ga = plsc.load_gather(a_v, [row_idx, col_idx])
                gb = plsc.load_gather(b_v, [row_idx, col_idx])
                acc = acc + ga * gb
            o_v[pl.ds(h * chunk + g * _L, _L)] = acc
            return carry

        lax.fori_loop(0, chunk // _L, group, 0)

    pltpu.sync_copy(o_v, out_hbm.at[pl.ds(base, rows)])


def kernel(gum, gim):
    n_rows, n_cols = gum.shape
    rows_per_w = n_rows // _NW
    mesh = plsc.VectorSubcoreMesh(core_axis_name="c", subcore_axis_name="s")
    xui = pl.kernel(
        _sc_body,
        out_type=jax.ShapeDtypeStruct((n_rows,), jnp.float32),
        mesh=mesh,
        compiler_params=pltpu.CompilerParams(needs_layout_passes=False),
        scratch_types=[
            pltpu.VMEM((_CHUNK, n_cols), jnp.float32),
            pltpu.VMEM((_CHUNK, n_cols), jnp.float32),
            pltpu.VMEM((rows_per_w,), jnp.float32),
        ],
    )(gum, gim)
    return (xui, gum, gim)


# speedup vs baseline: 1.5869x; 1.5869x over previous
import jax
import jax.numpy as jnp
from jax import lax
from jax.experimental import pallas as pl
from jax.experimental.pallas import tpu as pltpu
from jax.experimental.pallas import tpu_sc as plsc

_NC = 2
_NS = 16
_NW = _NC * _NS
_L = 16
_CHUNK = 256


def _sc_body(gum_hbm, gim_hbm, out_hbm, a_v, b_v, o_v):
    rows = o_v.shape[0]
    chunk = a_v.shape[0]
    n_cols = a_v.shape[1]
    wid = lax.axis_index("s") * _NC + lax.axis_index("c")
    base = wid * rows

    lanes = lax.iota(jnp.int32, _L)

    for h in range(rows // chunk):
        pltpu.sync_copy(gum_hbm.at[pl.ds(base + h * chunk, chunk), :], a_v)
        pltpu.sync_copy(gim_hbm.at[pl.ds(base + h * chunk, chunk), :], b_v)

        def group(g, carry):
            row_idx = g * _L + lanes
            acc = jnp.zeros((_L,), jnp.float32)
            for c in range(n_cols):
                col_idx = jnp.bitwise_and(c + lanes, n_cols - 1)
                ga = plsc.load_gather(a_v, [row_idx, col_idx])
                gb = plsc.load_gather(b_v, [row_idx, col_idx])
                acc = acc + ga * gb
            o_v[pl.ds(h * chunk + g * _L, _L)] = acc
            return carry

        lax.fori_loop(0, chunk // _L, group, 0)

    pltpu.sync_copy(o_v, out_hbm.at[pl.ds(base, rows)])


def kernel(gum, gim):
    n_rows, n_cols = gum.shape
    rows_per_w = n_rows // _NW
    mesh = plsc.VectorSubcoreMesh(core_axis_name="c", subcore_axis_name="s")
    xui = pl.kernel(
        _sc_body,
        out_type=jax.ShapeDtypeStruct((n_rows,), jnp.float32),
        mesh=mesh,
        compiler_params=pltpu.CompilerParams(needs_layout_passes=False),
        scratch_types=[
            pltpu.VMEM((_CHUNK, n_cols), jnp.float32),
            pltpu.VMEM((_CHUNK, n_cols), jnp.float32),
            pltpu.VMEM((rows_per_w,), jnp.float32),
        ],
    )(gum, gim)
    return (xui, gum, gim)
